# baseline (speedup 1.0000x reference)
import jax
import jax.numpy as jnp
from jax.experimental import pallas as pl
from jax.experimental.pallas import tpu as pltpu

CHANNELS = 768
N_CLUSTERS = 512
N_TOKENS = 32768
BT = 2048
CH = 128
N_BLOCKS = N_TOKENS // BT


def _cluster_body(x_ref, w_ref, b_ref, out_ref, oh_ref, cnt_ref):
    i = pl.program_id(0)

    @pl.when(i == 0)
    def _init():
        out_ref[...] = jnp.zeros_like(out_ref)
        cnt_ref[...] = jnp.zeros_like(cnt_ref)

    w = w_ref[...]
    bias = b_ref[...]

    def _chunk(h, carry):
        xb = x_ref[pl.ds(h * CH, CH), :]
        logits = (
            jax.lax.dot_general(
                xb, w, (((1,), (1,)), ((), ())),
                preferred_element_type=jnp.float32,
            )
            + bias
        )
        rowmax = jnp.max(logits, axis=1, keepdims=True)
        oh_ref[pl.ds(h * CH, CH), :] = (logits == rowmax).astype(jnp.bfloat16)
        return carry

    jax.lax.fori_loop(0, BT // CH, _chunk, 0)

    onehot = oh_ref[...]
    out_ref[...] += jax.lax.dot_general(
        onehot,
        x_ref[...].astype(jnp.bfloat16),
        (((0,), (0,)), ((), ())),
        preferred_element_type=jnp.float32,
    )
    cnt_ref[...] += jnp.sum(onehot.astype(jnp.float32), axis=0, keepdims=True)

    @pl.when(i == N_BLOCKS - 1)
    def _finalize():
        out_ref[...] = out_ref[...] / cnt_ref[...].T


@jax.jit
def kernel(x, W, b):
    out = pl.pallas_call(
        _cluster_body,
        grid=(N_BLOCKS,),
        in_specs=[
            pl.BlockSpec((BT, CHANNELS), lambda i: (i, 0)),
            pl.BlockSpec((N_CLUSTERS, CHANNELS), lambda i: (0, 0)),
            pl.BlockSpec((1, N_CLUSTERS), lambda i: (0, 0)),
        ],
        out_specs=pl.BlockSpec((N_CLUSTERS, CHANNELS), lambda i: (0, 0)),
        out_shape=jax.ShapeDtypeStruct((N_CLUSTERS, CHANNELS), jnp.float32),
        scratch_shapes=[
            pltpu.VMEM((BT, N_CLUSTERS), jnp.bfloat16),
            pltpu.VMEM((1, N_CLUSTERS), jnp.float32),
        ],
    )(x, W, b.reshape(1, N_CLUSTERS))
    return out


# speedup vs baseline: 1.7098x; 1.7098x over previous
import jax
import jax.numpy as jnp
from jax.experimental import pallas as pl
from jax.experimental.pallas import tpu as pltpu

CHANNELS = 768
N_CLUSTERS = 512
N_TOKENS = 32768
BT = 2048
NB = 4
NC = N_CLUSTERS // NB
N_BLOCKS = N_TOKENS // BT


def _cluster_body(x_ref, w_ref, b_ref, out_ref, lg_ref, cnt_ref):
    i = pl.program_id(0)

    @pl.when(i == 0)
    def _init():
        out_ref[...] = jnp.zeros_like(out_ref)
        cnt_ref[...] = jnp.zeros_like(cnt_ref)

    xb = x_ref[...]
    rowmax = None
    for n in range(NB):
        wn = w_ref[pl.ds(n * NC, NC), :]
        ln = (
            jax.lax.dot_general(
                xb, wn, (((1,), (1,)), ((), ())),
                preferred_element_type=jnp.float32,
            )
            + b_ref[:, pl.ds(n * NC, NC)]
        )
        lg_ref[:, pl.ds(n * NC, NC)] = ln
        mn = jnp.max(ln, axis=1, keepdims=True)
        rowmax = mn if rowmax is None else jnp.maximum(rowmax, mn)

    onehot = (lg_ref[...] == rowmax).astype(jnp.bfloat16)
    out_ref[...] += jax.lax.dot_general(
        onehot,
        xb.astype(jnp.bfloat16),
        (((0,), (0,)), ((), ())),
        preferred_element_type=jnp.float32,
    )
    cnt_ref[...] += jnp.sum(onehot.astype(jnp.float32), axis=0, keepdims=True)

    @pl.when(i == N_BLOCKS - 1)
    def _finalize():
        out_ref[...] = out_ref[...] / cnt_ref[...].T


@jax.jit
def kernel(x, W, b):
    out = pl.pallas_call(
        _cluster_body,
        grid=(N_BLOCKS,),
        in_specs=[
            pl.BlockSpec((BT, CHANNELS), lambda i: (i, 0)),
            pl.BlockSpec((N_CLUSTERS, CHANNELS), lambda i: (0, 0)),
            pl.BlockSpec((1, N_CLUSTERS), lambda i: (0, 0)),
        ],
        out_specs=pl.BlockSpec((N_CLUSTERS, CHANNELS), lambda i: (0, 0)),
        out_shape=jax.ShapeDtypeStruct((N_CLUSTERS, CHANNELS), jnp.float32),
        scratch_shapes=[
            pltpu.VMEM((BT, N_CLUSTERS), jnp.float32),
            pltpu.VMEM((1, N_CLUSTERS), jnp.float32),
        ],
    )(x, W, b.reshape(1, N_CLUSTERS))
    return out


# speedup vs baseline: 2.2405x; 1.3104x over previous
import jax
import jax.numpy as jnp
from jax.experimental import pallas as pl
from jax.experimental.pallas import tpu as pltpu

CHANNELS = 768
N_CLUSTERS = 512
N_TOKENS = 32768
BT = 2048
N_BLOCKS = N_TOKENS // BT


def _cluster_body(x_ref, w_ref, b_ref, out_ref, cnt_ref):
    i = pl.program_id(0)

    @pl.when(i == 0)
    def _init():
        out_ref[...] = jnp.zeros_like(out_ref)
        cnt_ref[...] = jnp.zeros_like(cnt_ref)

    xb = x_ref[...]
    logits = (
        jnp.dot(xb, w_ref[...].T, preferred_element_type=jnp.float32)
        + b_ref[...]
    )
    rowmax = jnp.max(logits, axis=1, keepdims=True)
    mask = logits == rowmax
    onehot = mask.astype(jnp.bfloat16)
    out_ref[...] += jax.lax.dot_general(
        onehot,
        xb.astype(jnp.bfloat16),
        (((0,), (0,)), ((), ())),
        preferred_element_type=jnp.float32,
    )
    cnt_ref[...] += jnp.sum(mask.astype(jnp.float32), axis=0, keepdims=True)

    @pl.when(i == N_BLOCKS - 1)
    def _finalize():
        out_ref[...] = out_ref[...] / cnt_ref[...].T


@jax.jit
def kernel(x, W, b):
    out = pl.pallas_call(
        _cluster_body,
        grid=(N_BLOCKS,),
        in_specs=[
            pl.BlockSpec((BT, CHANNELS), lambda i: (i, 0)),
            pl.BlockSpec((N_CLUSTERS, CHANNELS), lambda i: (0, 0)),
            pl.BlockSpec((1, N_CLUSTERS), lambda i: (0, 0)),
        ],
        out_specs=pl.BlockSpec((N_CLUSTERS, CHANNELS), lambda i: (0, 0)),
        out_shape=jax.ShapeDtypeStruct((N_CLUSTERS, CHANNELS), jnp.float32),
        scratch_shapes=[pltpu.VMEM((1, N_CLUSTERS), jnp.float32)],
    )(x, W, b.reshape(1, N_CLUSTERS))
    return out
